# baseline (speedup 1.0000x reference)
import functools
import math

import jax
import jax.numpy as jnp
from jax import lax
from jax.experimental import pallas as pl
from jax.experimental.pallas import tpu as pltpu
from jax.experimental.pallas import tpu_sc as plsc

DIM = 64
LANES = 16
CHUNK = 128
GBUF = 2
OBUF = 2
SCALE = math.sqrt(DIM)

ROWS_PER_IT = 4


def _scale_chunk(src, bg, dst, bo, off_v, j):
    def group(i, carry):
        g0 = i * LANES
        ov = off_v[j, pl.ds(g0, LANES)]
        for sub in range(LANES // ROWS_PER_IT):
            vals = []
            for dr in range(ROWS_PER_IT):
                r = g0 + sub * ROWS_PER_IT + dr
                off = ov[sub * ROWS_PER_IT + dr]
                for k in range(DIM // LANES):
                    sl = pl.ds(off + k * LANES, LANES)
                    vals.append((r, k, src[bg, r, sl] * SCALE))
            for r, k, v in vals:
                dst[bo, r, pl.ds(k * LANES, LANES)] = v
        return carry

    lax.fori_loop(0, CHUNK // LANES, group, 0)


@functools.lru_cache(maxsize=None)
def _make_gather(NW, NC, n_chunks, b_per_w, B):
    mesh = plsc.VectorSubcoreMesh(core_axis_name="c", subcore_axis_name="s")

    @functools.partial(
        pl.kernel,
        out_type=jax.ShapeDtypeStruct((B, DIM), jnp.float32),
        mesh=mesh,
        compiler_params=pltpu.CompilerParams(
            use_tc_tiling_on_sc=True, needs_layout_passes=False
        ),
        scratch_types=[
            pltpu.VMEM((n_chunks, CHUNK), jnp.int32),
            pltpu.VMEM((n_chunks, CHUNK), jnp.int32),
            pltpu.VMEM((GBUF, CHUNK, 2 * DIM), jnp.float32),
            pltpu.VMEM((OBUF, CHUNK, DIM), jnp.float32),
            pltpu.SemaphoreType.DMA,
            pltpu.SemaphoreType.DMA,
            pltpu.SemaphoreType.DMA,
            pltpu.SemaphoreType.DMA,
        ],
    )
    def body(lid_hbm, off_hbm, table_hbm, out_hbm, lid_v, off_v, bufg, bufo,
             sg0, sg1, so0, so1):
        semg = (sg0, sg1)
        semo = (so0, so1)
        wid = lax.axis_index("s") * NC + lax.axis_index("c")
        base = wid * b_per_w
        pltpu.sync_copy(lid_hbm.at[wid], lid_v)
        pltpu.sync_copy(off_hbm.at[wid], off_v)

        def g_start(j, b):
            pltpu.async_copy(table_hbm.at[lid_v.at[j]], bufg.at[b], semg[b])

        def g_wait(j, b):
            pltpu.make_async_copy(
                table_hbm.at[lid_v.at[j]], bufg.at[b], semg[b]
            ).wait()

        def o_start(j, b):
            pltpu.async_copy(
                bufo.at[b], out_hbm.at[pl.ds(base + j * CHUNK, CHUNK)], semo[b]
            )

        def o_wait(j, b):
            pltpu.make_async_copy(
                bufo.at[b], out_hbm.at[pl.ds(base + j * CHUNK, CHUNK)], semo[b]
            ).wait()

        for b in range(GBUF):
            g_start(b, b)

        for j in range(GBUF):
            g_wait(j, j % GBUF)
            if j >= OBUF:
                o_wait(j - OBUF, j % OBUF)
            _scale_chunk(bufg, j % GBUF, bufo, j % OBUF, off_v, j)
            g_start(j + GBUF, j % GBUF)
            o_start(j, j % OBUF)

        def outer(i, carry):
            for b in range(GBUF):
                j = i * GBUF + b
                bo = b % OBUF
                g_wait(j, b)
                o_wait(j - OBUF, bo)
                _scale_chunk(bufg, b, bufo, bo, off_v, j)
                g_start(j + GBUF, b)
                o_start(j, bo)
            return carry

        lax.fori_loop(1, n_chunks // GBUF - 1, outer, 0)

        for t in range(GBUF):
            j = n_chunks - GBUF + t
            g_wait(j, j % GBUF)
            o_wait(j - OBUF, (j - OBUF) % OBUF)
            _scale_chunk(bufg, j % GBUF, bufo, j % OBUF, off_v, j)
            o_start(j, j % OBUF)
        for t in range(OBUF):
            j = n_chunks - OBUF + t
            o_wait(j, j % OBUF)

    return body


def kernel(x, table):
    batch, seq = x.shape
    B = batch * seq
    info = plsc.get_sparse_core_info()
    NC, NS = info.num_cores, info.num_subcores
    NW = NC * NS
    b_per_w = B // NW
    n_chunks = b_per_w // CHUNK
    idx = x.reshape(NW, n_chunks, CHUNK).astype(jnp.int32)
    lid = idx >> 1
    off = (idx & 1) * DIM
    table2 = table.reshape(table.shape[0] // 2, 2 * DIM)
    out = _make_gather(NW, NC, n_chunks, b_per_w, B)(lid, off, table2)
    return out.reshape(batch, seq, DIM)


# speedup vs baseline: 1.0833x; 1.0833x over previous
import functools
import math

import jax
import jax.numpy as jnp
from jax import lax
from jax.experimental import pallas as pl
from jax.experimental.pallas import tpu as pltpu
from jax.experimental.pallas import tpu_sc as plsc

DIM = 64
LANES = 16
CHUNK = 128
GBUF = 4
OBUF = 2
SCALE = math.sqrt(DIM)

ROWS_PER_IT = 4


def _scale_chunk(src, bg, dst, bo):
    def rows(i, carry):
        r0 = i * ROWS_PER_IT
        vals = []
        for dr in range(ROWS_PER_IT):
            for k in range(DIM // LANES):
                sl = pl.ds(k * LANES, LANES)
                vals.append((dr, sl, src[bg, r0 + dr, sl] * SCALE))
        for dr, sl, v in vals:
            dst[bo, r0 + dr, sl] = v
        return carry

    lax.fori_loop(0, CHUNK // ROWS_PER_IT, rows, 0)


@functools.lru_cache(maxsize=None)
def _make_gather(NW, NC, n_chunks, b_per_w, B):
    mesh = plsc.VectorSubcoreMesh(core_axis_name="c", subcore_axis_name="s")

    @functools.partial(
        pl.kernel,
        out_type=jax.ShapeDtypeStruct((B, DIM), jnp.float32),
        mesh=mesh,
        compiler_params=pltpu.CompilerParams(
            use_tc_tiling_on_sc=True, needs_layout_passes=False
        ),
        scratch_types=[
            pltpu.VMEM((n_chunks, CHUNK), jnp.int32),
            pltpu.VMEM((GBUF, CHUNK, 2 * DIM), jnp.float32),
            pltpu.VMEM((OBUF, CHUNK, DIM), jnp.float32),
            pltpu.SemaphoreType.DMA,
            pltpu.SemaphoreType.DMA,
            pltpu.SemaphoreType.DMA,
            pltpu.SemaphoreType.DMA,
            pltpu.SemaphoreType.DMA,
            pltpu.SemaphoreType.DMA,
        ],
    )
    def body(idx_hbm, table_hbm, out_hbm, idx_v, bufg, bufo,
             sg0, sg1, sg2, sg3, so0, so1):
        semg = (sg0, sg1, sg2, sg3)
        semo = (so0, so1)
        wid = lax.axis_index("s") * NC + lax.axis_index("c")
        base = wid * b_per_w
        pltpu.sync_copy(idx_hbm.at[wid], idx_v)

        def g_start(j, b):
            pltpu.async_copy(table_hbm.at[idx_v.at[j]], bufg.at[b], semg[b])

        def g_wait(j, b):
            pltpu.make_async_copy(
                table_hbm.at[idx_v.at[j]], bufg.at[b], semg[b]
            ).wait()

        def o_start(j, b):
            pltpu.async_copy(
                bufo.at[b], out_hbm.at[pl.ds(base + j * CHUNK, CHUNK)], semo[b]
            )

        def o_wait(j, b):
            pltpu.make_async_copy(
                bufo.at[b], out_hbm.at[pl.ds(base + j * CHUNK, CHUNK)], semo[b]
            ).wait()

        for b in range(GBUF):
            g_start(b, b)

        for j in range(GBUF):
            g_wait(j, j % GBUF)
            if j >= OBUF:
                o_wait(j - OBUF, j % OBUF)
            _scale_chunk(bufg, j % GBUF, bufo, j % OBUF)
            g_start(j + GBUF, j % GBUF)
            o_start(j, j % OBUF)

        def outer(i, carry):
            for b in range(GBUF):
                j = i * GBUF + b
                bo = b % OBUF
                g_wait(j, b)
                o_wait(j - OBUF, bo)
                _scale_chunk(bufg, b, bufo, bo)
                g_start(j + GBUF, b)
                o_start(j, bo)
            return carry

        lax.fori_loop(1, n_chunks // GBUF - 1, outer, 0)

        for t in range(GBUF):
            j = n_chunks - GBUF + t
            g_wait(j, j % GBUF)
            o_wait(j - OBUF, (j - OBUF) % OBUF)
            _scale_chunk(bufg, j % GBUF, bufo, j % OBUF)
            o_start(j, j % OBUF)
        for t in range(OBUF):
            j = n_chunks - OBUF + t
            o_wait(j, j % OBUF)

    return body


def kernel(x, table):
    batch, seq = x.shape
    B = batch * seq
    info = plsc.get_sparse_core_info()
    NC, NS = info.num_cores, info.num_subcores
    NW = NC * NS
    b_per_w = B // NW
    n_chunks = b_per_w // CHUNK
    idx = x.reshape(NW, n_chunks, CHUNK).astype(jnp.int32)
    table2 = jnp.pad(table, ((0, 0), (0, DIM)))
    out = _make_gather(NW, NC, n_chunks, b_per_w, B)(idx, table2)
    return out.reshape(batch, seq, DIM)
